# Initial kernel scaffold; baseline (speedup 1.0000x reference)
import jax
import jax.numpy as jnp
from jax.experimental import pallas as pl


def kernel(loc_pred, conf_pred, targets, anchors):
    raise NotImplementedError("write your pallas kernel here")


# speedup vs baseline: 4.2912x; 4.2912x over previous
import functools

import jax
import jax.numpy as jnp
from jax.experimental import pallas as pl
from jax.experimental.pallas import tpu as pltpu

_NEGPOS_RATIO = 3.0
_THRESHOLD = 0.5
_NL = 128


def _smooth_l1_sum(d):
    ad = jnp.abs(d)
    return jnp.sum(jnp.where(ad < 1.0, 0.5 * ad * ad, ad - 0.5))


def _match_body(num_anchors, tgt_ref, anc_ref, loc_ref, lab_ref, stats_ref):
    P = num_anchors
    NO = tgt_ref.shape[1]
    NR = anc_ref.shape[1]

    ax = anc_ref[0]
    ay = anc_ref[1]
    aw = anc_ref[2]
    ah = anc_ref[3]
    ax2 = ax + aw
    ay2 = ay + ah
    area_a = (ax2 - ax) * (ay2 - ay)

    i0 = jax.lax.broadcasted_iota(jnp.int32, (NR, _NL), 0)
    i1 = jax.lax.broadcasted_iota(jnp.int32, (NR, _NL), 1)
    flat = i0 * _NL + i1
    valid = flat < P

    tgt = tgt_ref[0]
    tx_v = tgt[:, 0][None, :]
    ty_v = tgt[:, 1][None, :]
    tx2_v = tgt[:, 2][None, :]
    ty2_v = tgt[:, 3][None, :]
    tl_v = tgt[:, 4][None, :]
    tw_v = tx2_v - tx_v
    th_v = ty2_v - ty_v
    log_tw = jnp.log(tw_v)
    log_th = jnp.log(th_v)

    def _sc(a, t):
        return jnp.sum(a[0:1, t:t + 1])

    bto = jnp.full((NR, _NL), -1.0, jnp.float32)
    bti = jnp.zeros((NR, _NL), jnp.int32)
    bpis = []
    for t in range(NO):
        tx, ty, tx2, ty2 = (_sc(tx_v, t), _sc(ty_v, t), _sc(tx2_v, t), _sc(ty2_v, t))
        area_t = (tx2 - tx) * (ty2 - ty)
        w = jnp.clip(jnp.minimum(tx2, ax2) - jnp.maximum(tx, ax), 0.0, None)
        h = jnp.clip(jnp.minimum(ty2, ay2) - jnp.maximum(ty, ay), 0.0, None)
        inter = w * h
        ov = inter / (area_t + area_a - inter)
        upd = ov > bto
        bti = jnp.where(upd, t, bti)
        bto = jnp.where(upd, ov, bto)
        mx = jnp.max(jnp.where(valid, ov, -1.0))
        bpis.append(jnp.min(jnp.where((ov == mx) & valid, flat, P)))

    scat = [jnp.zeros((NR, _NL), jnp.float32) for _ in range(5)]
    for t in range(NO):
        m = flat == bpis[t]
        pr_x = jnp.sum(jnp.where(m, ax, 0.0))
        pr_y = jnp.sum(jnp.where(m, ay, 0.0))
        pr_w = jnp.sum(jnp.where(m, aw, 0.0))
        pr_h = jnp.sum(jnp.where(m, ah, 0.0))
        log_pr_w = jnp.sum(jnp.where(m, jnp.log(jnp.where(m, aw, 1.0)), 0.0))
        log_pr_h = jnp.sum(jnp.where(m, jnp.log(jnp.where(m, ah, 1.0)), 0.0))
        f = [(_sc(tx_v, t) - pr_x) / pr_w,
             (_sc(ty_v, t) - pr_y) / pr_h,
             _sc(log_tw, t) - log_pr_w,
             _sc(log_th, t) - log_pr_h,
             _sc(tl_v, t)]
        scat = [jnp.where(m, f[j], scat[j]) for j in range(5)]

    g = [jnp.zeros((NR, _NL), jnp.float32) for _ in range(5)]
    for t in range(NO):
        tm = bti == t
        vals = [_sc(tx_v, t), _sc(ty_v, t), _sc(tw_v, t), _sc(th_v, t), _sc(tl_v, t)]
        g = [jnp.where(tm, vals[j], g[j]) for j in range(5)]

    second = [(g[0] - ax) / aw, (g[1] - ay) / ah,
              jnp.log(g[2]) - jnp.log(aw), jnp.log(g[3]) - jnp.log(ah)]

    over = bto > _THRESHOLD
    lab = jnp.where(over, g[4], scat[4])
    lab = jnp.where(valid, lab, 0.0)
    pos = lab > 0.0

    locsum = jnp.float32(0.0)
    for j in range(4):
        locT_j = jnp.where(over, second[j], scat[j])
        locsum = locsum + _smooth_l1_sum(
            jnp.where(pos, loc_ref[0, j] - locT_j, 0.0))
    npos = jnp.sum(jnp.where(pos, 1.0, 0.0))

    lab_ref[0] = lab
    iota8 = jax.lax.broadcasted_iota(jnp.int32, (8, 1), 0)
    stats_ref[0] = jnp.where(iota8 == 0, npos,
                             jnp.where(iota8 == 1, locsum, 0.0))


def _conf_body(num_anchors, chunk, conf_ref, lab_ref, cls_ref, stats_ref):
    pc = pl.program_id(1)
    Pc = conf_ref.shape[1]
    C = conf_ref.shape[2]

    rows = pc * chunk + jax.lax.broadcasted_iota(jnp.int32, (Pc, 1), 0)
    valid = rows < num_anchors

    cf = conf_ref[0]
    lab = lab_ref[0]
    pos = valid & (lab > 0.0)
    posf = pos.astype(jnp.float32)

    cmax = jnp.max(cf, axis=1, keepdims=True)
    lse = jnp.log(jnp.sum(jnp.exp(cf - cmax), axis=1, keepdims=True)) + cmax
    c0 = cf[:, 0:1]
    iota_c = jax.lax.broadcasted_iota(jnp.int32, (Pc, C), 1)
    clab = jnp.sum(jnp.where(iota_c == lab.astype(jnp.int32), cf, 0.0),
                   axis=1, keepdims=True)

    cls_ref[0] = jnp.where(valid, (lse - c0) * (1.0 - posf), 0.0)
    possum = jnp.sum(jnp.where(pos, lse - clab, 0.0))

    iota8 = jax.lax.broadcasted_iota(jnp.int32, (8, 1), 0)
    col = jnp.where(iota8 == 0, possum, 0.0)

    @pl.when(pc == 0)
    def _():
        stats_ref[0] = col

    @pl.when(pc != 0)
    def _():
        stats_ref[0] = stats_ref[0] + col


def _mine_body(num_classes, cls_ref, statm_ref, statc_ref, out_ref):
    B, P = cls_ref.shape
    cls = cls_ref[...]
    npos = statm_ref[...][:, 0:1]
    locsum = statm_ref[...][:, 1:2]
    possum = statc_ref[...][:, 0:1]
    k = jnp.minimum(_NEGPOS_RATIO * npos, float(P - 1))

    bits = jax.lax.bitcast_convert_type(cls, jnp.int32)

    def body(i, t):
        bit = 30 - i
        cand = t | jax.lax.shift_left(jnp.int32(1), bit)
        cnt = jnp.sum((bits >= cand).astype(jnp.float32), axis=1, keepdims=True)
        return jnp.where(cnt >= k, cand, t)

    t = jax.lax.fori_loop(0, 31, body, jnp.zeros((B, 1), jnp.int32))
    v = jax.lax.bitcast_convert_type(t, jnp.float32)
    gt = bits > t
    cntgt = jnp.sum(gt.astype(jnp.float32), axis=1, keepdims=True)
    sumgt = jnp.sum(jnp.where(gt, cls, 0.0), axis=1, keepdims=True)
    m = k - cntgt
    negsum = sumgt + jnp.where(m > 0, m * v, 0.0)

    clsloss = possum + negsum + (P - (npos + k)) * jnp.log(float(num_classes))
    N = jnp.sum(npos)
    out0 = jnp.sum(locsum) / N
    out1 = jnp.sum(clsloss) / N
    iota_s = jax.lax.broadcasted_iota(jnp.int32, (1, 128), 1)
    out_ref[...] = jnp.where(iota_s == 0, out0, jnp.where(iota_s == 1, out1, 0.0))


def kernel(loc_pred, conf_pred, targets, anchors):
    B, P, C = conf_pred.shape
    NO = targets.shape[1]
    NR = (P + _NL - 1) // _NL
    PAD = NR * _NL

    anc = jnp.pad(anchors.T, ((0, 0), (0, PAD - P))).reshape(4, NR, _NL)
    loc4 = jnp.pad(jnp.transpose(loc_pred, (0, 2, 1)),
                   ((0, 0), (0, 0), (0, PAD - P))).reshape(B, 4, NR, _NL)

    lab_t, stats_m = pl.pallas_call(
        functools.partial(_match_body, P),
        grid=(B,),
        in_specs=[
            pl.BlockSpec((1, NO, 5), lambda b: (b, 0, 0)),
            pl.BlockSpec((4, NR, _NL), lambda b: (0, 0, 0)),
            pl.BlockSpec((1, 4, NR, _NL), lambda b: (b, 0, 0, 0)),
        ],
        out_specs=[
            pl.BlockSpec((1, NR, _NL), lambda b: (b, 0, 0)),
            pl.BlockSpec((1, 8, 1), lambda b: (b, 0, 0)),
        ],
        out_shape=[
            jax.ShapeDtypeStruct((B, NR, _NL), jnp.float32),
            jax.ShapeDtypeStruct((B, 8, 1), jnp.float32),
        ],
    )(targets, anc, loc4)

    lab = lab_t.reshape(B, PAD)[:, :P].reshape(B, P, 1)

    chunk = 1096
    nch = (P + chunk - 1) // chunk
    cls, stats_c = pl.pallas_call(
        functools.partial(_conf_body, P, chunk),
        grid=(B, nch),
        in_specs=[
            pl.BlockSpec((1, chunk, C), lambda b, c: (b, c, 0)),
            pl.BlockSpec((1, chunk, 1), lambda b, c: (b, c, 0)),
        ],
        out_specs=[
            pl.BlockSpec((1, chunk, 1), lambda b, c: (b, c, 0)),
            pl.BlockSpec((1, 8, 1), lambda b, c: (b, 0, 0)),
        ],
        out_shape=[
            jax.ShapeDtypeStruct((B, P, 1), jnp.float32),
            jax.ShapeDtypeStruct((B, 8, 1), jnp.float32),
        ],
    )(conf_pred, lab)

    out = pl.pallas_call(
        functools.partial(_mine_body, C),
        grid=(1,),
        in_specs=[
            pl.BlockSpec((B, P), lambda i: (0, 0)),
            pl.BlockSpec((B, 8), lambda i: (0, 0)),
            pl.BlockSpec((B, 8), lambda i: (0, 0)),
        ],
        out_specs=pl.BlockSpec((1, 128), lambda i: (0, 0)),
        out_shape=jax.ShapeDtypeStruct((1, 128), jnp.float32),
    )(cls.reshape(B, P), stats_m.reshape(B, 8), stats_c.reshape(B, 8))

    return out[0, 0], out[0, 1]
